# baseline (speedup 1.0000x reference)
import functools

import jax
import jax.numpy as jnp
from jax import lax
from jax.experimental import pallas as pl
from jax.experimental.pallas import tpu as pltpu
from jax.experimental.pallas import tpu_sc as plsc

_NC = 2
_NS = 16
_NW = _NC * _NS
_L = 16

_B = 16384
_NEG = 20
_D = 64
_DV = _D // _L
_V = 1000000
_VP = _V // 2
_DP = 2 * _D
_BW = _B // _NW
_CB = 16
_NCHUNK = _BW // _CB
_SBLK = 8
_NROWS = 1 + _NEG


def _sc_scores_kernel(tt_hbm, ct_hbm, tidx_hbm, cidx_hbm, nidx_hbm,
                      scores_hbm,
                      tidx_v, cidx_v, nidx_v,
                      tbufA, cbufA, nbufA, ptA, pcA, pnA,
                      tbufB, cbufB, nbufB, ptB, pcB, pnB,
                      scores_v, semA, semB):
    wid = lax.axis_index("s") * _NC + lax.axis_index("c")
    base = wid * _BW

    pltpu.sync_copy(tidx_hbm.at[pl.ds(base, _BW)], tidx_v.at[pl.ds(0, _BW)])
    pltpu.sync_copy(cidx_hbm.at[pl.ds(base, _BW)], cidx_v.at[pl.ds(0, _BW)])
    pltpu.sync_copy(nidx_hbm.at[pl.ds(base * _NEG, _BW * _NEG)],
                    nidx_v.at[pl.ds(0, _BW * _NEG)])

    bufs = [(tbufA, cbufA, nbufA, ptA, pcA, pnA, semA),
            (tbufB, cbufB, nbufB, ptB, pcB, pnB, semB)]

    def issue(g, b):
        tb, cb, nb, pt, pc, pn, sem = bufs[b]
        col0 = g * _CB
        pt[...] = jnp.right_shift(tidx_v[pl.ds(col0, _CB)], 1)
        pc[...] = jnp.right_shift(cidx_v[pl.ds(col0, _CB)], 1)
        ncol0 = col0 * _NEG

        def stage_n(k2, carry):
            v = jnp.right_shift(nidx_v[pl.ds(ncol0 + k2 * _L, _L)], 1)
            pn[pl.ds(k2 * _L, _L)] = v
            return carry

        lax.fori_loop(0, _CB * _NEG // _L, stage_n, 0)

        pltpu.async_copy(tt_hbm.at[pt], tb, sem)
        pltpu.async_copy(ct_hbm.at[pc], cb, sem)
        pltpu.async_copy(ct_hbm.at[pn], nb, sem)

    def drain(b):
        tb, cb, nb, _, _, _, sem = bufs[b]
        pltpu.make_async_copy(tt_hbm.at[pl.ds(0, _CB)], tb, sem).wait()
        pltpu.make_async_copy(tt_hbm.at[pl.ds(0, _CB)], cb, sem).wait()
        pltpu.make_async_copy(ct_hbm.at[pl.ds(0, _CB * _NEG)], nb,
                              sem).wait()

    lanes = lax.iota(jnp.int32, _L)

    def compute(g, b):
        tb, cb, nb, _, _, _, _ = bufs[b]

        def elem_body(i, accs):
            sel = lanes == i
            col = g * _CB + i
            toff = (tidx_v[pl.ds(col, _L)][0] & 1) * _D
            coff = (cidx_v[pl.ds(col, _L)][0] & 1) * _D
            ts = [tb[i, pl.ds(toff + j * _L, _L)] for j in range(_DV)]
            cs = [cb[i, pl.ds(coff + j * _L, _L)] for j in range(_DV)]
            p = ts[0] * cs[0]
            for j in range(1, _DV):
                p = p + ts[j] * cs[j]
            out = [jnp.where(sel, -jnp.sum(p), accs[0])]
            nrow = i * _NEG
            ncol = col * _NEG
            for k in range(_NEG):
                noff = (nidx_v[pl.ds(ncol + k, _L)][0] & 1) * _D
                q = ts[0] * nb[nrow + k, pl.ds(noff, _L)]
                for j in range(1, _DV):
                    q = q + ts[j] * nb[nrow + k, pl.ds(noff + j * _L, _L)]
                out.append(jnp.where(sel, jnp.sum(q), accs[1 + k]))
            return tuple(out)

        accs = lax.fori_loop(
            0, _L, elem_body,
            tuple(jnp.zeros((_L,), jnp.float32) for _ in range(_NROWS)))
        col = pl.ds((g % _SBLK) * _CB, _L)
        for r in range(_NROWS):
            scores_v[r, col] = accs[r]

    def flush(g):
        blk = (g // _SBLK) * (_SBLK * _CB)
        pltpu.sync_copy(scores_v,
                        scores_hbm.at[:, pl.ds(base + blk, _SBLK * _CB)])

    def pair_body(p, carry):
        c0 = p * 2
        c1 = c0 + 1
        issue(c1, 1)
        drain(0)
        compute(c0, 0)

        @pl.when(p < _NCHUNK // 2 - 1)
        def _():
            issue(c1 + 1, 0)

        drain(1)
        compute(c1, 1)

        @pl.when(c1 % _SBLK == _SBLK - 1)
        def _():
            flush(c1)

        return carry

    issue(0, 0)
    lax.fori_loop(0, _NCHUNK // 2, pair_body, 0)


def _sc_scores(target_idx, context_idx, neg_idx_flat, t2, c2):
    mesh = plsc.VectorSubcoreMesh(core_axis_name="c", subcore_axis_name="s")
    kern = functools.partial(
        pl.kernel,
        mesh=mesh,
        compiler_params=pltpu.CompilerParams(needs_layout_passes=False),
        out_type=jax.ShapeDtypeStruct((_NROWS, _B), jnp.float32),
        scratch_types=[
            pltpu.VMEM((_BW + _L,), jnp.int32),
            pltpu.VMEM((_BW + _L,), jnp.int32),
            pltpu.VMEM((_BW * _NEG + _L,), jnp.int32),
            pltpu.VMEM((_CB, _DP), jnp.float32),
            pltpu.VMEM((_CB, _DP), jnp.float32),
            pltpu.VMEM((_CB * _NEG, _DP), jnp.float32),
            pltpu.VMEM((_CB,), jnp.int32),
            pltpu.VMEM((_CB,), jnp.int32),
            pltpu.VMEM((_CB * _NEG,), jnp.int32),
            pltpu.VMEM((_CB, _DP), jnp.float32),
            pltpu.VMEM((_CB, _DP), jnp.float32),
            pltpu.VMEM((_CB * _NEG, _DP), jnp.float32),
            pltpu.VMEM((_CB,), jnp.int32),
            pltpu.VMEM((_CB,), jnp.int32),
            pltpu.VMEM((_CB * _NEG,), jnp.int32),
            pltpu.VMEM((_NROWS, _SBLK * _CB), jnp.float32),
            pltpu.SemaphoreType.DMA,
            pltpu.SemaphoreType.DMA,
        ],
    )(_sc_scores_kernel)
    return kern(t2, c2, target_idx, context_idx, neg_idx_flat)


def _tc_loss_kernel(scores_ref, out_ref):
    x = scores_ref[...]
    sp = jnp.maximum(x, 0.0) + jnp.log1p(jnp.exp(-jnp.abs(x)))
    out_ref[...] = jnp.full((1, 1), jnp.sum(sp) * (1.0 / _B), jnp.float32)


def _tc_loss(scores2d):
    out = pl.pallas_call(
        _tc_loss_kernel,
        out_shape=jax.ShapeDtypeStruct((1, 1), jnp.float32),
    )(scores2d)
    return out[0, 0]


def kernel(target_idx, context_idx, neg_idx, target_table, context_table):
    target_idx = target_idx.astype(jnp.int32)
    context_idx = context_idx.astype(jnp.int32)
    neg_idx_flat = neg_idx.astype(jnp.int32).reshape(_B * _NEG)
    t2 = target_table.reshape(_VP, _DP)
    c2 = context_table.reshape(_VP, _DP)
    scores = _sc_scores(target_idx, context_idx, neg_idx_flat, t2, c2)
    return _tc_loss(scores)
